# baseline (speedup 1.0000x reference)
import functools

import jax
import jax.numpy as jnp
from jax import lax
from jax.experimental import pallas as pl
from jax.experimental.pallas import tpu as pltpu
from jax.experimental.pallas import tpu_sc as plsc

N_NODES = 10000
N_EDGES = 320000
D = 128
N_REL = 500
NREL_PAD = 512

NC = 2
NS = 16
NW = NC * NS
L = 16

B = 40
NCHUNK = N_EDGES // B
NIT = NCHUNK // NW
G = 10
NG = NIT // G
EPW = N_EDGES // NW
GB3 = G * 3 * B
T_ROWS = 2 * N_NODES + NREL_PAD
assert NCHUNK % NW == 0 and NIT % G == 0 and G % 2 == 0
N_ACC = 10240
ROWS_PER_SUB = N_ACC // NS
NODE_BLK = 1000


def _proj_nodes_body(hid_ref, ent_ref, wm_ref, wh_ref, bh_ref,
                     pvi_ref, pvj_ref, pre_ref):
    hid = hid_ref[...]
    wm = wm_ref[...]
    wh = wh_ref[...]
    pvi_ref[...] = jnp.dot(hid, wm[0:D, :], preferred_element_type=jnp.float32)
    pvj_ref[...] = jnp.dot(hid, wm[2 * D:3 * D, :],
                           preferred_element_type=jnp.float32)
    pre_ref[...] = (
        jnp.dot(hid, wh[D:2 * D, :], preferred_element_type=jnp.float32)
        + jnp.dot(ent_ref[...], wh[2 * D:3 * D, :],
                  preferred_element_type=jnp.float32)
        + bh_ref[...]
    )


def _proj_rel_body(rel_ref, wm_ref, bm_ref, prel_ref):
    prel_ref[...] = (
        jnp.dot(rel_ref[...], wm_ref[...][D:2 * D, :],
                preferred_element_type=jnp.float32)
        + bm_ref[...]
    )


def _project_tables(hid2d, ent, relp, Wm, bm, Wh, bh):
    nblk = N_NODES // NODE_BLK
    pvi, pvj, pre = pl.pallas_call(
        _proj_nodes_body,
        grid=(nblk,),
        in_specs=[
            pl.BlockSpec((NODE_BLK, D), lambda i: (i, 0)),
            pl.BlockSpec((NODE_BLK, D), lambda i: (i, 0)),
            pl.BlockSpec((3 * D, D), lambda i: (0, 0)),
            pl.BlockSpec((3 * D, D), lambda i: (0, 0)),
            pl.BlockSpec((D,), lambda i: (0,)),
        ],
        out_specs=[
            pl.BlockSpec((NODE_BLK, D), lambda i: (i, 0)),
            pl.BlockSpec((NODE_BLK, D), lambda i: (i, 0)),
            pl.BlockSpec((NODE_BLK, D), lambda i: (i, 0)),
        ],
        out_shape=[
            jax.ShapeDtypeStruct((N_NODES, D), jnp.float32),
            jax.ShapeDtypeStruct((N_NODES, D), jnp.float32),
            jax.ShapeDtypeStruct((N_NODES, D), jnp.float32),
        ],
    )(hid2d, ent, Wm, Wh, bh)
    prel = pl.pallas_call(
        _proj_rel_body,
        out_shape=jax.ShapeDtypeStruct((NREL_PAD, D), jnp.float32),
    )(relp, Wm, bm)
    return pvi, pvj, pre, prel


_SC_MESH = plsc.VectorSubcoreMesh(
    core_axis_name="c", subcore_axis_name="s", num_cores=NC, num_subcores=NS)


@functools.partial(
    pl.kernel,
    out_type=[
        jax.ShapeDtypeStruct((NC, N_ACC, D), jnp.float32),
        jax.ShapeDtypeStruct((NC, NS, N_ACC), jnp.float32),
    ],
    mesh=_SC_MESH,
    compiler_params=pltpu.CompilerParams(needs_layout_passes=False),
    scratch_types=[
        pltpu.VMEM((GB3,), jnp.int32),
        pltpu.VMEM((G * B,), jnp.int32),
        pltpu.VMEM((2, B), jnp.int32),
        pltpu.VMEM((2, 3 * B, D), jnp.float32),
        pltpu.VMEM((N_ACC,), jnp.float32),
        pltpu.VMEM_SHARED((N_ACC, D), jnp.float32),
        pltpu.SemaphoreType.DMA,
        pltpu.SemaphoreType.DMA,
        pltpu.SemaphoreType.DMA,
    ],
)
def _sc_edge_kernel(gidx_hbm, vj_hbm, tab_hbm, out_hbm, out_cnt_hbm,
                    gidx_blk, vj_blk, vj_scat, rows, cnt_local,
                    acc, isem, gsem, ssem):
    c = lax.axis_index("c")
    s = lax.axis_index("s")
    wid = s * NC + c
    zero16 = jnp.zeros((L,), jnp.float32)
    ones16 = jnp.ones((L,), jnp.float32)
    tail_mask = lax.iota(jnp.int32, L) >= (L - B % L)

    def zrow(r, carry):
        for k in range(D // L):
            rows[0, r, pl.ds(k * L, L)] = zero16
        return carry

    lax.fori_loop(0, B, zrow, 0)
    base_row = s * ROWS_PER_SUB
    for k in range(ROWS_PER_SUB // B):
        pltpu.sync_copy(rows.at[0, pl.ds(0, B)],
                        acc.at[pl.ds(base_row + k * B, B)])

    def zcnt(r, carry):
        cnt_local[pl.ds(r * L, L)] = zero16
        return carry

    lax.fori_loop(0, N_ACC // L, zcnt, 0)
    plsc.subcore_barrier()

    def issue_gather(j, sl):
        pltpu.async_copy(tab_hbm.at[gidx_blk.at[pl.ds(j * 3 * B, 3 * B)]],
                         rows.at[sl], gsem)

    def wait_gather(j, sl):
        pltpu.make_async_copy(
            tab_hbm.at[gidx_blk.at[pl.ds(j * 3 * B, 3 * B)]],
            rows.at[sl], gsem).wait()

    def fill_scat_idx(j, sl):
        for off in (0, L, B - L):
            vj_scat[sl, pl.ds(off, L)] = vj_blk[pl.ds(j * B + off, L)]

    def drain_scatter(sl):
        pltpu.make_async_copy(rows.at[sl, pl.ds(0, B)], acc.at[vj_scat.at[sl]],
                              ssem).wait()

    def group_body(g, carry):
        gedge = wid * EPW + g * (G * B)
        ci0 = pltpu.async_copy(
            gidx_hbm.at[pl.ds(3 * gedge, GB3)], gidx_blk, isem)
        ci1 = pltpu.async_copy(vj_hbm.at[pl.ds(gedge, G * B)], vj_blk, isem)
        ci0.wait()
        ci1.wait()
        issue_gather(0, 0)

        def pair_body(p, inner):
            for sl in (0, 1):
                j = p * 2 + sl
                nxt = 1 - sl
                wait_gather(j, sl)

                @pl.when(j >= 1)
                def _():
                    drain_scatter(nxt)

                @pl.when(j + 1 < G)
                def _():
                    issue_gather(j + 1, nxt)

                fill_scat_idx(j, sl)

                def erow(e, icarry):
                    for k in range(D // L):
                        lanes = pl.ds(k * L, L)
                        x = (rows[sl, e, lanes] + rows[sl, B + e, lanes]
                             + rows[sl, 2 * B + e, lanes])
                        ex = jnp.exp(x + x)
                        rows[sl, e, lanes] = 1.0 - 2.0 / (ex + 1.0)
                    return icarry

                lax.fori_loop(0, B, erow, 0)
                pltpu.async_copy(rows.at[sl, pl.ds(0, B)],
                                 acc.at[vj_scat.at[sl]], ssem, add=True)

                for k in range(B // L):
                    plsc.addupdate_scatter(
                        cnt_local, [vj_scat[sl, pl.ds(k * L, L)]], ones16)
                if B % L:
                    plsc.addupdate_scatter(
                        cnt_local, [vj_scat[sl, pl.ds(B - L, L)]], ones16,
                        mask=tail_mask)
            return inner

        lax.fori_loop(0, G // 2, pair_body, 0)
        drain_scatter(1)
        return carry

    lax.fori_loop(0, NG, group_body, 0)

    pltpu.sync_copy(cnt_local, out_cnt_hbm.at[c, s])

    plsc.subcore_barrier()
    pltpu.sync_copy(acc.at[pl.ds(base_row, ROWS_PER_SUB)],
                    out_hbm.at[c, pl.ds(base_row, ROWS_PER_SUB)])


def _node_update_body(parts_ref, cnt_ref, hid_ref, pre_ref, wh_ref, out_ref):
    aggr = parts_ref[0] + parts_ref[1]
    ev = aggr * lax.rsqrt(cnt_ref[...])
    u = jnp.tanh(
        jnp.dot(ev, wh_ref[...][0:D, :], preferred_element_type=jnp.float32)
        + pre_ref[...])
    out_ref[...] = hid_ref[...] + u


def _node_update(parts, cnt, hid2d, pre, Wh):
    nblk = N_NODES // NODE_BLK
    return pl.pallas_call(
        _node_update_body,
        grid=(nblk,),
        in_specs=[
            pl.BlockSpec((NC, NODE_BLK, D), lambda i: (0, i, 0)),
            pl.BlockSpec((NODE_BLK, 1), lambda i: (i, 0)),
            pl.BlockSpec((NODE_BLK, D), lambda i: (i, 0)),
            pl.BlockSpec((NODE_BLK, D), lambda i: (i, 0)),
            pl.BlockSpec((3 * D, D), lambda i: (0, 0)),
        ],
        out_specs=pl.BlockSpec((NODE_BLK, D), lambda i: (i, 0)),
        out_shape=jax.ShapeDtypeStruct((N_NODES, D), jnp.float32),
    )(parts, cnt, hid2d, pre, Wh)


def kernel(inputs, selected_edges, relation_emb, entity_emb, Wm, bm, Wh, bh):
    hid2d = inputs[0]
    vi = selected_edges[:, 1]
    vj = selected_edges[:, 2]
    rel = selected_edges[:, 3]
    gidx = jnp.concatenate(
        [vi.reshape(NCHUNK, B),
         vj.reshape(NCHUNK, B) + N_NODES,
         rel.reshape(NCHUNK, B) + 2 * N_NODES], axis=1).reshape(-1)
    relp = jnp.pad(relation_emb, ((0, NREL_PAD - N_REL), (0, 0)))
    pvi, pvj, pre, prel = _project_tables(
        hid2d, entity_emb, relp, Wm, bm, Wh, bh)
    tab = jnp.concatenate([pvi, pvj, prel], axis=0)
    parts, cnt_parts = _sc_edge_kernel(gidx, vj, tab)
    cnt = cnt_parts.sum(axis=(0, 1))[:N_NODES, None]
    out2d = _node_update(parts, cnt, hid2d, pre, Wh)
    return out2d[None]


# speedup vs baseline: 1.2307x; 1.2307x over previous
import functools

import jax
import jax.numpy as jnp
from jax import lax
from jax.experimental import pallas as pl
from jax.experimental.pallas import tpu as pltpu
from jax.experimental.pallas import tpu_sc as plsc

N_NODES = 10000
N_EDGES = 320000
D = 128
N_REL = 500
NREL_PAD = 512

NC = 2
NS = 16
NW = NC * NS
L = 16

B = 40
NCHUNK = N_EDGES // B
NIT = NCHUNK // NW
G = 50
NG = NIT // G
EPW = N_EDGES // NW
assert NCHUNK % NW == 0 and NIT % G == 0 and G % 2 == 0
N_ACC = 10240
ROWS_PER_SUB = N_ACC // NS
NODE_BLK = 1000


def _proj_nodes_body(hid_ref, wm_ref, pvi_ref, pvj_ref):
    hid = hid_ref[...]
    wm = wm_ref[...]
    pvi_ref[...] = jnp.dot(hid, wm[0:D, :], preferred_element_type=jnp.float32)
    pvj_ref[...] = jnp.dot(hid, wm[2 * D:3 * D, :],
                           preferred_element_type=jnp.float32)


def _pre_body(hid_ref, ent_ref, wh_ref, bh_ref, pre_ref):
    pre_ref[...] = (
        jnp.dot(hid_ref[...], wh_ref[...][D:2 * D, :],
                preferred_element_type=jnp.float32)
        + jnp.dot(ent_ref[...], wh_ref[...][2 * D:3 * D, :],
                  preferred_element_type=jnp.float32)
        + bh_ref[...]
    )


def _proj_rel_body(rel_ref, wm_ref, bm_ref, prel_ref):
    prel_ref[...] = (
        jnp.dot(rel_ref[...], wm_ref[...][D:2 * D, :],
                preferred_element_type=jnp.float32)
        + bm_ref[...]
    )


def _project_tables(hid2d, relp, Wm, bm):
    nblk = N_NODES // NODE_BLK
    pvi, pvj = pl.pallas_call(
        _proj_nodes_body,
        grid=(nblk,),
        in_specs=[
            pl.BlockSpec((NODE_BLK, D), lambda i: (i, 0)),
            pl.BlockSpec((3 * D, D), lambda i: (0, 0)),
        ],
        out_specs=[
            pl.BlockSpec((NODE_BLK, D), lambda i: (i, 0)),
            pl.BlockSpec((NODE_BLK, D), lambda i: (i, 0)),
        ],
        out_shape=[
            jax.ShapeDtypeStruct((N_NODES, D), jnp.float32),
            jax.ShapeDtypeStruct((N_NODES, D), jnp.float32),
        ],
    )(hid2d, Wm)
    prel = pl.pallas_call(
        _proj_rel_body,
        out_shape=jax.ShapeDtypeStruct((NREL_PAD, D), jnp.float32),
    )(relp, Wm, bm)
    return pvi, pvj, prel


def _node_pre(hid2d, ent, Wh, bh):
    nblk = N_NODES // NODE_BLK
    return pl.pallas_call(
        _pre_body,
        grid=(nblk,),
        in_specs=[
            pl.BlockSpec((NODE_BLK, D), lambda i: (i, 0)),
            pl.BlockSpec((NODE_BLK, D), lambda i: (i, 0)),
            pl.BlockSpec((3 * D, D), lambda i: (0, 0)),
            pl.BlockSpec((D,), lambda i: (0,)),
        ],
        out_specs=pl.BlockSpec((NODE_BLK, D), lambda i: (i, 0)),
        out_shape=jax.ShapeDtypeStruct((N_NODES, D), jnp.float32),
    )(hid2d, ent, Wh, bh)


_SC_MESH = plsc.VectorSubcoreMesh(
    core_axis_name="c", subcore_axis_name="s", num_cores=NC, num_subcores=NS)


@functools.partial(
    pl.kernel,
    out_type=[
        jax.ShapeDtypeStruct((NC, N_ACC, D), jnp.float32),
        jax.ShapeDtypeStruct((NC, NS, N_ACC), jnp.float32),
    ],
    mesh=_SC_MESH,
    compiler_params=pltpu.CompilerParams(needs_layout_passes=False),
    scratch_types=[
        pltpu.VMEM((G * B,), jnp.int32),
        pltpu.VMEM((G * B,), jnp.int32),
        pltpu.VMEM((G * B,), jnp.int32),
        pltpu.VMEM((2, B), jnp.int32),
        pltpu.VMEM((2, B, D), jnp.float32),
        pltpu.VMEM((2, B, D), jnp.float32),
        pltpu.VMEM((2, B, D), jnp.float32),
        pltpu.VMEM((N_ACC,), jnp.float32),
        pltpu.VMEM_SHARED((N_ACC, D), jnp.float32),
        pltpu.SemaphoreType.DMA,
        pltpu.SemaphoreType.DMA,
        pltpu.SemaphoreType.DMA,
        pltpu.SemaphoreType.DMA,
        pltpu.SemaphoreType.DMA,
    ],
)
def _sc_edge_kernel(vi_hbm, vj_hbm, rel_hbm, pvi_hbm, pvj_hbm, prel_hbm,
                    out_hbm, out_cnt_hbm, vi_blk, vj_blk, rel_blk, vj_scat,
                    rows_vi, rows_vj, rows_rel, cnt_local,
                    acc, isem, gsem0, gsem1, gsem2, ssem):
    c = lax.axis_index("c")
    s = lax.axis_index("s")
    wid = s * NC + c
    zero16 = jnp.zeros((L,), jnp.float32)
    ones16 = jnp.ones((L,), jnp.float32)
    tail_mask = lax.iota(jnp.int32, L) >= (L - B % L)

    def zrow(r, carry):
        for k in range(D // L):
            rows_vi[0, r, pl.ds(k * L, L)] = zero16
        return carry

    lax.fori_loop(0, B, zrow, 0)
    base_row = s * ROWS_PER_SUB
    for k in range(ROWS_PER_SUB // B):
        pltpu.sync_copy(rows_vi.at[0], acc.at[pl.ds(base_row + k * B, B)])

    def zcnt(r, carry):
        cnt_local[pl.ds(r * L, L)] = zero16
        return carry

    lax.fori_loop(0, N_ACC // L, zcnt, 0)
    plsc.subcore_barrier()

    def issue_gather(j, sl):
        pltpu.async_copy(pvi_hbm.at[vi_blk.at[pl.ds(j * B, B)]],
                         rows_vi.at[sl], gsem0)
        pltpu.async_copy(pvj_hbm.at[vj_blk.at[pl.ds(j * B, B)]],
                         rows_vj.at[sl], gsem1)
        pltpu.async_copy(prel_hbm.at[rel_blk.at[pl.ds(j * B, B)]],
                         rows_rel.at[sl], gsem2)

    def wait_gather(j, sl):
        pltpu.make_async_copy(pvi_hbm.at[vi_blk.at[pl.ds(j * B, B)]],
                              rows_vi.at[sl], gsem0).wait()
        pltpu.make_async_copy(pvj_hbm.at[vj_blk.at[pl.ds(j * B, B)]],
                              rows_vj.at[sl], gsem1).wait()
        pltpu.make_async_copy(prel_hbm.at[rel_blk.at[pl.ds(j * B, B)]],
                              rows_rel.at[sl], gsem2).wait()

    def fill_scat_idx(j, sl):
        for off in (0, L, B - L):
            vj_scat[sl, pl.ds(off, L)] = vj_blk[pl.ds(j * B + off, L)]

    def drain_scatter(sl):
        pltpu.make_async_copy(rows_vi.at[sl], acc.at[vj_scat.at[sl]],
                              ssem).wait()

    def group_body(g, carry):
        gedge = wid * EPW + g * (G * B)
        ci0 = pltpu.async_copy(vi_hbm.at[pl.ds(gedge, G * B)], vi_blk, isem)
        ci1 = pltpu.async_copy(vj_hbm.at[pl.ds(gedge, G * B)], vj_blk, isem)
        ci2 = pltpu.async_copy(rel_hbm.at[pl.ds(gedge, G * B)], rel_blk, isem)
        ci0.wait()
        ci1.wait()
        ci2.wait()
        issue_gather(0, 0)

        def pair_body(p, inner):
            for sl in (0, 1):
                j = p * 2 + sl
                nxt = 1 - sl
                wait_gather(j, sl)

                @pl.when(j >= 1)
                def _():
                    drain_scatter(nxt)

                @pl.when(j + 1 < G)
                def _():
                    issue_gather(j + 1, nxt)

                fill_scat_idx(j, sl)

                def erow(e, icarry):
                    for k in range(D // L):
                        lanes = pl.ds(k * L, L)
                        x = (rows_vi[sl, e, lanes] + rows_vj[sl, e, lanes]
                             + rows_rel[sl, e, lanes])
                        ex = jnp.exp(x + x)
                        rows_vi[sl, e, lanes] = 1.0 - 2.0 / (ex + 1.0)
                    return icarry

                lax.fori_loop(0, B, erow, 0)
                pltpu.async_copy(rows_vi.at[sl], acc.at[vj_scat.at[sl]], ssem,
                                 add=True)

                for k in range(B // L):
                    plsc.addupdate_scatter(
                        cnt_local, [vj_scat[sl, pl.ds(k * L, L)]], ones16)
                if B % L:
                    plsc.addupdate_scatter(
                        cnt_local, [vj_scat[sl, pl.ds(B - L, L)]], ones16,
                        mask=tail_mask)
            return inner

        lax.fori_loop(0, G // 2, pair_body, 0)
        drain_scatter(1)
        return carry

    lax.fori_loop(0, NG, group_body, 0)

    pltpu.sync_copy(cnt_local, out_cnt_hbm.at[c, s])

    plsc.subcore_barrier()
    pltpu.sync_copy(acc.at[pl.ds(base_row, ROWS_PER_SUB)],
                    out_hbm.at[c, pl.ds(base_row, ROWS_PER_SUB)])


def _node_update_body(parts_ref, cnt_ref, hid_ref, pre_ref, wh_ref, out_ref):
    aggr = parts_ref[0] + parts_ref[1]
    ev = aggr * lax.rsqrt(cnt_ref[...])
    u = jnp.tanh(
        jnp.dot(ev, wh_ref[...][0:D, :], preferred_element_type=jnp.float32)
        + pre_ref[...])
    out_ref[...] = hid_ref[...] + u


def _node_update(parts, cnt, hid2d, pre, Wh):
    nblk = N_NODES // NODE_BLK
    return pl.pallas_call(
        _node_update_body,
        grid=(nblk,),
        in_specs=[
            pl.BlockSpec((NC, NODE_BLK, D), lambda i: (0, i, 0)),
            pl.BlockSpec((NODE_BLK, 1), lambda i: (i, 0)),
            pl.BlockSpec((NODE_BLK, D), lambda i: (i, 0)),
            pl.BlockSpec((NODE_BLK, D), lambda i: (i, 0)),
            pl.BlockSpec((3 * D, D), lambda i: (0, 0)),
        ],
        out_specs=pl.BlockSpec((NODE_BLK, D), lambda i: (i, 0)),
        out_shape=jax.ShapeDtypeStruct((N_NODES, D), jnp.float32),
    )(parts, cnt, hid2d, pre, Wh)


def kernel(inputs, selected_edges, relation_emb, entity_emb, Wm, bm, Wh, bh):
    hid2d = inputs[0]
    vi = selected_edges[:, 1]
    vj = selected_edges[:, 2]
    rel = selected_edges[:, 3]
    relp = jnp.pad(relation_emb, ((0, NREL_PAD - N_REL), (0, 0)))
    pvi, pvj, prel = _project_tables(hid2d, relp, Wm, bm)
    parts, cnt_parts = _sc_edge_kernel(vi, vj, rel, pvi, pvj, prel)
    pre = _node_pre(hid2d, entity_emb, Wh, bh)
    cnt = cnt_parts.sum(axis=(0, 1))[:N_NODES, None]
    out2d = _node_update(parts, cnt, hid2d, pre, Wh)
    return out2d[None]


# speedup vs baseline: 1.2401x; 1.0076x over previous
import functools

import jax
import jax.numpy as jnp
from jax import lax
from jax.experimental import pallas as pl
from jax.experimental.pallas import tpu as pltpu
from jax.experimental.pallas import tpu_sc as plsc

N_NODES = 10000
N_EDGES = 320000
D = 128
N_REL = 500
NREL_PAD = 512

NC = 2
NS = 16
NW = NC * NS
L = 16

B = 40
NCHUNK = N_EDGES // B
NIT = NCHUNK // NW
G = 50
NG = NIT // G
EPW = N_EDGES // NW
assert NCHUNK % NW == 0 and NIT % G == 0 and G % 2 == 0
N_ACC = 10240
ROWS_PER_SUB = N_ACC // NS
NODE_BLK = 1000


def _proj_nodes_body(hid_ref, wm_ref, pvi_ref, pvj_ref):
    hid = hid_ref[...]
    wm = wm_ref[...]
    pvi_ref[...] = jnp.dot(hid, wm[0:D, :], preferred_element_type=jnp.float32)
    pvj_ref[...] = jnp.dot(hid, wm[2 * D:3 * D, :],
                           preferred_element_type=jnp.float32)


def _pre_body(hid_ref, ent_ref, wh_ref, bh_ref, pre_ref):
    pre_ref[...] = (
        jnp.dot(hid_ref[...], wh_ref[...][D:2 * D, :],
                preferred_element_type=jnp.float32)
        + jnp.dot(ent_ref[...], wh_ref[...][2 * D:3 * D, :],
                  preferred_element_type=jnp.float32)
        + bh_ref[...]
    )


def _proj_rel_body(rel_ref, wm_ref, bm_ref, prel_ref):
    prel_ref[...] = (
        jnp.dot(rel_ref[...], wm_ref[...][D:2 * D, :],
                preferred_element_type=jnp.float32)
        + bm_ref[...]
    )


def _project_tables(hid2d, relp, Wm, bm):
    nblk = N_NODES // NODE_BLK
    pvi, pvj = pl.pallas_call(
        _proj_nodes_body,
        grid=(nblk,),
        in_specs=[
            pl.BlockSpec((NODE_BLK, D), lambda i: (i, 0)),
            pl.BlockSpec((3 * D, D), lambda i: (0, 0)),
        ],
        out_specs=[
            pl.BlockSpec((NODE_BLK, D), lambda i: (i, 0)),
            pl.BlockSpec((NODE_BLK, D), lambda i: (i, 0)),
        ],
        out_shape=[
            jax.ShapeDtypeStruct((N_NODES, D), jnp.float32),
            jax.ShapeDtypeStruct((N_NODES, D), jnp.float32),
        ],
    )(hid2d, Wm)
    prel = pl.pallas_call(
        _proj_rel_body,
        out_shape=jax.ShapeDtypeStruct((NREL_PAD, D), jnp.float32),
    )(relp, Wm, bm)
    return pvi, pvj, prel


def _node_pre(hid2d, ent, Wh, bh):
    nblk = N_NODES // NODE_BLK
    return pl.pallas_call(
        _pre_body,
        grid=(nblk,),
        in_specs=[
            pl.BlockSpec((NODE_BLK, D), lambda i: (i, 0)),
            pl.BlockSpec((NODE_BLK, D), lambda i: (i, 0)),
            pl.BlockSpec((3 * D, D), lambda i: (0, 0)),
            pl.BlockSpec((D,), lambda i: (0,)),
        ],
        out_specs=pl.BlockSpec((NODE_BLK, D), lambda i: (i, 0)),
        out_shape=jax.ShapeDtypeStruct((N_NODES, D), jnp.float32),
    )(hid2d, ent, Wh, bh)


_SC_MESH = plsc.VectorSubcoreMesh(
    core_axis_name="c", subcore_axis_name="s", num_cores=NC, num_subcores=NS)


@functools.partial(
    pl.kernel,
    out_type=[
        jax.ShapeDtypeStruct((NC, N_ACC, D), jnp.float32),
        jax.ShapeDtypeStruct((NC, NS, N_ACC), jnp.float32),
    ],
    mesh=_SC_MESH,
    compiler_params=pltpu.CompilerParams(needs_layout_passes=False),
    scratch_types=[
        pltpu.VMEM((G * B,), jnp.int32),
        pltpu.VMEM((G * B,), jnp.int32),
        pltpu.VMEM((G * B,), jnp.int32),
        pltpu.VMEM((2, B), jnp.int32),
        pltpu.VMEM((2, B, D), jnp.float32),
        pltpu.VMEM((2, B, D), jnp.float32),
        pltpu.VMEM((2, B, D), jnp.float32),
        pltpu.VMEM((N_ACC,), jnp.float32),
        pltpu.VMEM_SHARED((N_ACC, D), jnp.float32),
        pltpu.SemaphoreType.DMA,
        pltpu.SemaphoreType.DMA,
        pltpu.SemaphoreType.DMA,
        pltpu.SemaphoreType.DMA,
        pltpu.SemaphoreType.DMA,
    ],
)
def _sc_edge_kernel(vi_hbm, vj_hbm, rel_hbm, pvi_hbm, pvj_hbm, prel_hbm,
                    out_hbm, out_cnt_hbm, vi_blk, vj_blk, rel_blk, vj_scat,
                    rows_vi, rows_vj, rows_rel, cnt_local,
                    acc, isem, gsem0, gsem1, gsem2, ssem):
    c = lax.axis_index("c")
    s = lax.axis_index("s")
    wid = s * NC + c
    zero16 = jnp.zeros((L,), jnp.float32)
    ones16 = jnp.ones((L,), jnp.float32)
    tail_mask = lax.iota(jnp.int32, L) >= (L - B % L)

    def zrow(r, carry):
        for k in range(D // L):
            rows_vi[0, r, pl.ds(k * L, L)] = zero16
        return carry

    lax.fori_loop(0, B, zrow, 0)
    base_row = s * ROWS_PER_SUB
    for k in range(ROWS_PER_SUB // B):
        pltpu.sync_copy(rows_vi.at[0], acc.at[pl.ds(base_row + k * B, B)])

    def zcnt(r, carry):
        cnt_local[pl.ds(r * L, L)] = zero16
        return carry

    lax.fori_loop(0, N_ACC // L, zcnt, 0)
    plsc.subcore_barrier()

    def issue_gather(j, sl):
        pltpu.async_copy(pvi_hbm.at[vi_blk.at[pl.ds(j * B, B)]],
                         rows_vi.at[sl], gsem0)
        pltpu.async_copy(pvj_hbm.at[vj_blk.at[pl.ds(j * B, B)]],
                         rows_vj.at[sl], gsem1)
        pltpu.async_copy(prel_hbm.at[rel_blk.at[pl.ds(j * B, B)]],
                         rows_rel.at[sl], gsem2)

    def wait_gather(j, sl):
        pltpu.make_async_copy(pvi_hbm.at[vi_blk.at[pl.ds(j * B, B)]],
                              rows_vi.at[sl], gsem0).wait()
        pltpu.make_async_copy(pvj_hbm.at[vj_blk.at[pl.ds(j * B, B)]],
                              rows_vj.at[sl], gsem1).wait()
        pltpu.make_async_copy(prel_hbm.at[rel_blk.at[pl.ds(j * B, B)]],
                              rows_rel.at[sl], gsem2).wait()

    def fill_scat_idx(j, sl):
        for off in (0, L, B - L):
            vj_scat[sl, pl.ds(off, L)] = vj_blk[pl.ds(j * B + off, L)]

    def drain_scatter(sl):
        pltpu.make_async_copy(rows_vi.at[sl], acc.at[vj_scat.at[sl]],
                              ssem).wait()

    def group_body(g, carry):
        gedge = wid * EPW + g * (G * B)
        ci0 = pltpu.async_copy(vi_hbm.at[pl.ds(gedge, G * B)], vi_blk, isem)
        ci1 = pltpu.async_copy(vj_hbm.at[pl.ds(gedge, G * B)], vj_blk, isem)
        ci2 = pltpu.async_copy(rel_hbm.at[pl.ds(gedge, G * B)], rel_blk, isem)
        ci0.wait()
        ci1.wait()
        ci2.wait()
        issue_gather(0, 0)

        def pair_body(p, inner):
            for sl in (0, 1):
                j = p * 2 + sl
                nxt = 1 - sl
                wait_gather(j, sl)

                @pl.when(j >= 1)
                def _():
                    drain_scatter(nxt)

                @pl.when(j + 1 < G)
                def _():
                    issue_gather(j + 1, nxt)

                fill_scat_idx(j, sl)

                def erow(h, icarry):
                    for dd in range(2):
                        e = h * 2 + dd
                        for k in range(D // L):
                            lanes = pl.ds(k * L, L)
                            x = (rows_vi[sl, e, lanes] + rows_vj[sl, e, lanes]
                                 + rows_rel[sl, e, lanes])
                            ex = jnp.exp(x + x)
                            rows_vi[sl, e, lanes] = 1.0 - 2.0 / (ex + 1.0)
                    return icarry

                lax.fori_loop(0, B // 2, erow, 0)
                pltpu.async_copy(rows_vi.at[sl], acc.at[vj_scat.at[sl]], ssem,
                                 add=True)

                for k in range(B // L):
                    plsc.addupdate_scatter(
                        cnt_local, [vj_scat[sl, pl.ds(k * L, L)]], ones16)
                if B % L:
                    plsc.addupdate_scatter(
                        cnt_local, [vj_scat[sl, pl.ds(B - L, L)]], ones16,
                        mask=tail_mask)
            return inner

        lax.fori_loop(0, G // 2, pair_body, 0)
        drain_scatter(1)
        return carry

    lax.fori_loop(0, NG, group_body, 0)

    pltpu.sync_copy(cnt_local, out_cnt_hbm.at[c, s])

    plsc.subcore_barrier()
    pltpu.sync_copy(acc.at[pl.ds(base_row, ROWS_PER_SUB)],
                    out_hbm.at[c, pl.ds(base_row, ROWS_PER_SUB)])


def _node_update_body(parts_ref, cnt_ref, hid_ref, pre_ref, wh_ref, out_ref):
    aggr = parts_ref[0] + parts_ref[1]
    ev = aggr * lax.rsqrt(cnt_ref[...])
    u = jnp.tanh(
        jnp.dot(ev, wh_ref[...][0:D, :], preferred_element_type=jnp.float32)
        + pre_ref[...])
    out_ref[...] = hid_ref[...] + u


def _node_update(parts, cnt, hid2d, pre, Wh):
    nblk = N_NODES // NODE_BLK
    return pl.pallas_call(
        _node_update_body,
        grid=(nblk,),
        in_specs=[
            pl.BlockSpec((NC, NODE_BLK, D), lambda i: (0, i, 0)),
            pl.BlockSpec((NODE_BLK, 1), lambda i: (i, 0)),
            pl.BlockSpec((NODE_BLK, D), lambda i: (i, 0)),
            pl.BlockSpec((NODE_BLK, D), lambda i: (i, 0)),
            pl.BlockSpec((3 * D, D), lambda i: (0, 0)),
        ],
        out_specs=pl.BlockSpec((NODE_BLK, D), lambda i: (i, 0)),
        out_shape=jax.ShapeDtypeStruct((N_NODES, D), jnp.float32),
    )(parts, cnt, hid2d, pre, Wh)


def kernel(inputs, selected_edges, relation_emb, entity_emb, Wm, bm, Wh, bh):
    hid2d = inputs[0]
    vi = selected_edges[:, 1]
    vj = selected_edges[:, 2]
    rel = selected_edges[:, 3]
    relp = jnp.pad(relation_emb, ((0, NREL_PAD - N_REL), (0, 0)))
    pvi, pvj, prel = _project_tables(hid2d, relp, Wm, bm)
    parts, cnt_parts = _sc_edge_kernel(vi, vj, rel, pvi, pvj, prel)
    pre = _node_pre(hid2d, entity_emb, Wh, bh)
    cnt = cnt_parts.sum(axis=(0, 1))[:N_NODES, None]
    out2d = _node_update(parts, cnt, hid2d, pre, Wh)
    return out2d[None]
